# Initial kernel scaffold; baseline (speedup 1.0000x reference)
import jax
import jax.numpy as jnp
from jax.experimental import pallas as pl


def kernel(h, index):
    raise NotImplementedError("write your pallas kernel here")


# speedup vs baseline: 6.3491x; 6.3491x over previous
import functools

import jax
import jax.numpy as jnp
from jax import lax
from jax.experimental import pallas as pl
from jax.experimental.pallas import tpu as pltpu
from jax.experimental.pallas import tpu_sc as plsc

N_ROWS = 320000
D = 128
N_SEG = 512
N_WORKERS = 32
ROWS_PER_W = N_ROWS // N_WORKERS
CHUNK = 80
CHUNKS_PER_W = ROWS_PER_W // CHUNK
SEG_PER_SUB = N_SEG // 16


def _sc_body(h_hbm, idx_hbm, out_hbm, buf0, buf1, idx_v, zero_v, acc_sh,
             sem0, sem1):
    core = lax.axis_index("c")
    sub = lax.axis_index("s")
    wid = core * 16 + sub
    row_base = wid * ROWS_PER_W

    def zrow(r, _):
        for k in range(D // 16):
            zero_v[r, pl.ds(k * 16, 16)] = jnp.zeros((16,), jnp.float32)
        return 0
    lax.fori_loop(0, SEG_PER_SUB, zrow, 0)
    pltpu.sync_copy(zero_v, acc_sh.at[pl.ds(sub * SEG_PER_SUB, SEG_PER_SUB)])

    pltpu.sync_copy(idx_hbm.at[wid], idx_v)

    plsc.subcore_barrier()

    pltpu.async_copy(h_hbm.at[pl.ds(row_base, CHUNK)], buf0, sem0)

    bufs = (buf0, buf1)
    sems = (sem0, sem1)

    def pair_body(i, _):
        for b in range(2):
            c = i * 2 + b
            buf, sem = bufs[b], sems[b]
            nbuf, nsem = bufs[1 - b], sems[1 - b]
            pltpu.async_copy(
                h_hbm.at[pl.ds(row_base + (c + 1) * CHUNK, CHUNK)],
                nbuf, nsem)
            pltpu.make_async_copy(
                h_hbm.at[pl.ds(row_base + c * CHUNK, CHUNK)], buf, sem).wait()
            pltpu.sync_copy(buf, acc_sh.at[idx_v.at[c]], add=True)
        return 0

    lax.fori_loop(0, CHUNKS_PER_W // 2, pair_body, 0)

    c_last = CHUNKS_PER_W - 1
    pltpu.make_async_copy(
        h_hbm.at[pl.ds(row_base + c_last * CHUNK, CHUNK)], buf0, sem0).wait()
    pltpu.sync_copy(buf0, acc_sh.at[idx_v.at[c_last]], add=True)

    plsc.subcore_barrier()

    pltpu.sync_copy(
        acc_sh.at[pl.ds(sub * SEG_PER_SUB, SEG_PER_SUB)],
        out_hbm.at[core, pl.ds(sub * SEG_PER_SUB, SEG_PER_SUB)])


_sc_segsum = functools.partial(
    pl.kernel,
    out_type=jax.ShapeDtypeStruct((2, N_SEG, D), jnp.float32),
    mesh=plsc.VectorSubcoreMesh(core_axis_name="c", subcore_axis_name="s"),
    scratch_types=[
        pltpu.VMEM((CHUNK, D), jnp.float32),
        pltpu.VMEM((CHUNK, D), jnp.float32),
        pltpu.VMEM((CHUNKS_PER_W, CHUNK), jnp.int32),
        pltpu.VMEM((SEG_PER_SUB, D), jnp.float32),
        pltpu.VMEM_SHARED((N_SEG, D), jnp.float32),
        pltpu.SemaphoreType.DMA,
        pltpu.SemaphoreType.DMA,
    ],
)(_sc_body)


def _merge_body(p_ref, o_ref):
    o_ref[...] = p_ref[0] + p_ref[1]


def _merge(partials):
    return pl.pallas_call(
        _merge_body,
        out_shape=jax.ShapeDtypeStruct((N_SEG, D), jnp.float32),
    )(partials)


@jax.jit
def kernel(h, index):
    idx = index.astype(jnp.int32).reshape(N_WORKERS, CHUNKS_PER_W, CHUNK)
    partials = _sc_segsum(h, idx)
    return _merge(partials)


# speedup vs baseline: 6.3540x; 1.0008x over previous
import functools

import jax
import jax.numpy as jnp
from jax import lax
from jax.experimental import pallas as pl
from jax.experimental.pallas import tpu as pltpu
from jax.experimental.pallas import tpu_sc as plsc

N_ROWS = 320000
D = 128
N_SEG = 512
N_WORKERS = 32
ROWS_PER_W = N_ROWS // N_WORKERS
CHUNK = 80
CHUNKS_PER_W = ROWS_PER_W // CHUNK
SEG_PER_SUB = N_SEG // 16
NBUF = 6


def _sc_body(h_hbm, idx_hbm, out_hbm, *sc):
    bufs = sc[:NBUF]
    idx_v, zero_v, acc_sh = sc[NBUF:NBUF + 3]
    lsems = sc[NBUF + 3:2 * NBUF + 3]
    ssems = sc[2 * NBUF + 3:]
    core = lax.axis_index("c")
    sub = lax.axis_index("s")
    wid = core * 16 + sub
    row_base = wid * ROWS_PER_W

    def zrow(r, _):
        for k in range(D // 16):
            zero_v[r, pl.ds(k * 16, 16)] = jnp.zeros((16,), jnp.float32)
        return 0
    lax.fori_loop(0, SEG_PER_SUB, zrow, 0)
    pltpu.sync_copy(zero_v, acc_sh.at[pl.ds(sub * SEG_PER_SUB, SEG_PER_SUB)])

    pltpu.sync_copy(idx_hbm.at[wid], idx_v)

    plsc.subcore_barrier()

    def load_start(c, b):
        pltpu.async_copy(
            h_hbm.at[pl.ds(row_base + c * CHUNK, CHUNK)], bufs[b], lsems[b])

    def load_wait(c, b):
        pltpu.make_async_copy(
            h_hbm.at[pl.ds(row_base + c * CHUNK, CHUNK)], bufs[b],
            lsems[b]).wait()

    def scat_start(c, b):
        pltpu.async_copy(bufs[b], acc_sh.at[idx_v.at[c]], ssems[b], add=True)

    def scat_wait(c, b):
        pltpu.make_async_copy(
            bufs[b], acc_sh.at[idx_v.at[c]], ssems[b]).wait()

    MAIN = (CHUNKS_PER_W // NBUF) * NBUF

    load_start(0, 0)
    load_start(1, 1)

    def six_body(i, _):
        for b in range(NBUF):
            c = i * NBUF + b
            bn = (b + 2) % NBUF
            @pl.when(c >= 4)
            def _():
                scat_wait(c - 4, bn)
            load_start(c + 2, bn)
            load_wait(c, b)
            scat_start(c, b)
        return 0

    lax.fori_loop(0, MAIN // NBUF, six_body, 0)

    for c in range(MAIN, CHUNKS_PER_W):
        if c + 2 < CHUNKS_PER_W:
            scat_wait(c - 4, (c + 2) % NBUF)
            load_start(c + 2, (c + 2) % NBUF)
        load_wait(c, c % NBUF)
        scat_start(c, c % NBUF)
    for c in range(CHUNKS_PER_W - NBUF, CHUNKS_PER_W):
        scat_wait(c, c % NBUF)

    plsc.subcore_barrier()

    pltpu.sync_copy(
        acc_sh.at[pl.ds(sub * SEG_PER_SUB, SEG_PER_SUB)],
        out_hbm.at[core, pl.ds(sub * SEG_PER_SUB, SEG_PER_SUB)])


_sc_segsum = functools.partial(
    pl.kernel,
    out_type=jax.ShapeDtypeStruct((2, N_SEG, D), jnp.float32),
    mesh=plsc.VectorSubcoreMesh(core_axis_name="c", subcore_axis_name="s"),
    scratch_types=(
        [pltpu.VMEM((CHUNK, D), jnp.float32) for _ in range(NBUF)]
        + [
            pltpu.VMEM((CHUNKS_PER_W, CHUNK), jnp.int32),
            pltpu.VMEM((SEG_PER_SUB, D), jnp.float32),
            pltpu.VMEM_SHARED((N_SEG, D), jnp.float32),
        ]
        + [pltpu.SemaphoreType.DMA for _ in range(2 * NBUF)]
    ),
)(_sc_body)


def _merge_body(p_ref, o_ref):
    o_ref[...] = p_ref[0] + p_ref[1]


def _merge(partials):
    return pl.pallas_call(
        _merge_body,
        out_shape=jax.ShapeDtypeStruct((N_SEG, D), jnp.float32),
    )(partials)


@jax.jit
def kernel(h, index):
    idx = index.astype(jnp.int32).reshape(N_WORKERS, CHUNKS_PER_W, CHUNK)
    partials = _sc_segsum(h, idx)
    return _merge(partials)
